# Initial kernel scaffold; baseline (speedup 1.0000x reference)
import jax
import jax.numpy as jnp
from jax.experimental import pallas as pl


def kernel(v1, v2, s1, s2, h1, h2):
    raise NotImplementedError("write your pallas kernel here")


# speedup vs baseline: 2.7396x; 2.7396x over previous
import functools

import numpy as np
import jax
import jax.numpy as jnp
from jax import lax
from jax.experimental import pallas as pl
from jax.experimental.pallas import tpu as pltpu
from jax.experimental.pallas import tpu_sc as plsc

_B, _N, _D = 4096, 2048, 8192
_N1, _N2 = 64, 128
_NC, _NS = 2, 16
_NW = _NC * _NS
_RPW = _B // _NW
_L = 16

_R = 128
_G = _B // _R


def _dft(n):
    k = np.arange(n)
    ang = -2.0 * np.pi * np.outer(k, k) / n
    return np.cos(ang).astype(np.float32), np.sin(ang).astype(np.float32)


_F1R, _F1I = _dft(_N1)
_F2R, _F2I = _dft(_N2)
_ang = -2.0 * np.pi * np.outer(np.arange(_N1), np.arange(_N2)) / _D
_TWR = np.cos(_ang).astype(np.float32)
_TWI = np.sin(_ang).astype(np.float32)


def _sc_body(v1h, v2h, s1h, s2h, h1h, h2h, sk1h, sk2h,
             hv1, hv2, sv1, sv2, vrow, acc):
    wid = lax.axis_index("s") * _NC + lax.axis_index("c")
    base = wid * _RPW
    pltpu.sync_copy(h1h, hv1)
    pltpu.sync_copy(h2h, hv2)
    pltpu.sync_copy(s1h, sv1)
    pltpu.sync_copy(s2h, sv2)

    def do_row(vh, skh, hv, sv, row):
        pltpu.sync_copy(vh.at[row], vrow)

        @pl.loop(0, _D // _L)
        def _zero(i):
            acc[pl.ds(i * _L, _L)] = jnp.zeros((_L,), jnp.float32)

        @pl.loop(0, _N // _L)
        def _scat(j):
            idx = hv[pl.ds(j * _L, _L)]
            val = vrow[pl.ds(j * _L, _L)] * sv[pl.ds(j * _L, _L)]
            plsc.addupdate_scatter(acc, [idx], val)

        pltpu.sync_copy(acc, skh.at[row])

    @pl.loop(0, _RPW)
    def _rows(r):
        row = base + r
        do_row(v1h, sk1h, hv1, sv1, row)
        do_row(v2h, sk2h, hv2, sv2, row)


@functools.cache
def _sc_sketch():
    return pl.kernel(
        _sc_body,
        out_type=[jax.ShapeDtypeStruct((_B, _D), jnp.float32),
                  jax.ShapeDtypeStruct((_B, _D), jnp.float32)],
        mesh=plsc.VectorSubcoreMesh(core_axis_name="c", subcore_axis_name="s",
                                    num_cores=_NC, num_subcores=_NS),
        compiler_params=pltpu.CompilerParams(needs_layout_passes=False),
        scratch_types=[
            pltpu.VMEM((_N,), jnp.int32),
            pltpu.VMEM((_N,), jnp.int32),
            pltpu.VMEM((_N,), jnp.float32),
            pltpu.VMEM((_N,), jnp.float32),
            pltpu.VMEM((_N,), jnp.float32),
            pltpu.VMEM((_D,), jnp.float32),
        ],
    )


def _conv_body(x1_ref, x2_ref, f1r_ref, f1i_ref, f2r_ref, f2i_ref,
               twr_ref, twi_ref, y_ref, p_ref):
    f1r = f1r_ref[...]
    f1i = f1i_ref[...]
    f2r = f2r_ref[...]
    f2i = f2i_ref[...]
    twr = twr_ref[...][:, None, :]
    twi = twi_ref[...][:, None, :]

    def fwd(x):
        xt = jnp.transpose(x, (1, 0, 2)).reshape(_N1, _R * _N2)
        yr = jnp.dot(f1r, xt, preferred_element_type=jnp.float32)
        yi = jnp.dot(f1i, xt, preferred_element_type=jnp.float32)
        yr = yr.reshape(_N1, _R, _N2)
        yi = yi.reshape(_N1, _R, _N2)
        zr = (yr * twr - yi * twi).reshape(_N1 * _R, _N2)
        zi = (yr * twi + yi * twr).reshape(_N1 * _R, _N2)
        ur = jnp.dot(zr, f2r) - jnp.dot(zi, f2i)
        ui = jnp.dot(zr, f2i) + jnp.dot(zi, f2r)
        return ur, ui

    u1r, u1i = fwd(x1_ref[...])
    u2r, u2i = fwd(x2_ref[...])
    pr = u1r * u2r - u1i * u2i
    pi = u1r * u2i + u1i * u2r
    sr = jnp.dot(pr, f2r) + jnp.dot(pi, f2i)
    si = jnp.dot(pi, f2r) - jnp.dot(pr, f2i)
    sr = sr.reshape(_N1, _R, _N2)
    si = si.reshape(_N1, _R, _N2)
    tr = (sr * twr + si * twi).reshape(_N1, _R * _N2)
    ti = (si * twr - sr * twi).reshape(_N1, _R * _N2)
    xo = (jnp.dot(f1r, tr) + jnp.dot(f1i, ti)) * (1.0 / _D)
    x = jnp.transpose(xo.reshape(_N1, _R, _N2), (1, 0, 2))
    ax = jnp.abs(x)
    y_ref[...] = jnp.sign(x) * jnp.sqrt(ax)
    p_ref[...] = jnp.full((1, 1, 128), jnp.sum(ax) * (1.0 / 128.0), jnp.float32)


_conv = pl.pallas_call(
    _conv_body,
    grid=(_G,),
    in_specs=[
        pl.BlockSpec((_R, _N1, _N2), lambda g: (g, 0, 0)),
        pl.BlockSpec((_R, _N1, _N2), lambda g: (g, 0, 0)),
        pl.BlockSpec((_N1, _N1), lambda g: (0, 0)),
        pl.BlockSpec((_N1, _N1), lambda g: (0, 0)),
        pl.BlockSpec((_N2, _N2), lambda g: (0, 0)),
        pl.BlockSpec((_N2, _N2), lambda g: (0, 0)),
        pl.BlockSpec((_N1, _N2), lambda g: (0, 0)),
        pl.BlockSpec((_N1, _N2), lambda g: (0, 0)),
    ],
    out_specs=[
        pl.BlockSpec((_R, _N1, _N2), lambda g: (g, 0, 0)),
        pl.BlockSpec((1, 1, 128), lambda g: (g, 0, 0)),
    ],
    out_shape=[jax.ShapeDtypeStruct((_B, _N1, _N2), jnp.float32),
               jax.ShapeDtypeStruct((_G, 1, 128), jnp.float32)],
)


def _scale_body(y_ref, p_ref, o_ref):
    total = jnp.sum(p_ref[...])
    norm = jnp.sqrt(jnp.maximum(total, 1e-12))
    o_ref[...] = y_ref[...] * (1.0 / norm)


_scale = pl.pallas_call(
    _scale_body,
    grid=(_G,),
    in_specs=[
        pl.BlockSpec((_R, _D), lambda g: (g, 0)),
        pl.BlockSpec((_G, 1, 128), lambda g: (0, 0, 0)),
    ],
    out_specs=pl.BlockSpec((_R, _D), lambda g: (g, 0)),
    out_shape=jax.ShapeDtypeStruct((_B, _D), jnp.float32),
)


def kernel(v1, v2, s1, s2, h1, h2):
    sk1, sk2 = _sc_sketch()(v1, v2, s1, s2, h1, h2)
    y, parts = _conv(
        sk1.reshape(_B, _N1, _N2), sk2.reshape(_B, _N1, _N2),
        jnp.asarray(_F1R), jnp.asarray(_F1I),
        jnp.asarray(_F2R), jnp.asarray(_F2I),
        jnp.asarray(_TWR), jnp.asarray(_TWI),
    )
    return _scale(y.reshape(_B, _D), parts)


# speedup vs baseline: 4.0809x; 1.4896x over previous
import functools

import numpy as np
import jax
import jax.numpy as jnp
from jax import lax
from jax.experimental import pallas as pl
from jax.experimental.pallas import tpu as pltpu
from jax.experimental.pallas import tpu_sc as plsc

_B, _N, _D = 4096, 2048, 8192
_N1, _N2 = 64, 128
_NC, _NS = 2, 16
_NW = _NC * _NS
_RPW = _B // _NW
_L = 16

_R = 128
_G = _B // _R


def _dft(n):
    k = np.arange(n)
    ang = -2.0 * np.pi * np.outer(k, k) / n
    return np.cos(ang).astype(np.float32), np.sin(ang).astype(np.float32)


_F1R, _F1I = _dft(_N1)
_F2R, _F2I = _dft(_N2)
_ang = -2.0 * np.pi * np.outer(np.arange(_N1), np.arange(_N2)) / _D
_TWR = np.cos(_ang).astype(np.float32)
_TWI = np.sin(_ang).astype(np.float32)


def _sc_body(v1h, v2h, s1h, s2h, h1h, h2h, sk1h, sk2h,
             hv1, hv2, sv1, sv2, vb0, vb1, acc0, acc1,
             semv0, semv1, sema0, sema1):
    wid = lax.axis_index("s") * _NC + lax.axis_index("c")
    base = wid * _RPW
    pltpu.sync_copy(h1h, hv1)
    pltpu.sync_copy(h2h, hv2)
    pltpu.sync_copy(s1h, sv1)
    pltpu.sync_copy(s2h, sv2)
    vb = (vb0, vb1)
    acc = (acc0, acc1)
    semv = (semv0, semv1)
    sema = (sema0, sema1)

    for k in (0, 1):
        @pl.loop(0, _D // _L, unroll=8)
        def _z0(i, _k=k):
            acc[_k][pl.ds(i * _L, _L)] = jnp.zeros((_L,), jnp.float32)

    def zero_touched(accr, hv):
        @pl.loop(0, _N // _L, unroll=8)
        def _z(j):
            idx = hv[pl.ds(j * _L, _L)]
            plsc.store_scatter(accr, [idx], jnp.zeros((_L,), jnp.float32))

    def phase(vh, skh, hv, sv):
        for k in (0, 1):
            pltpu.make_async_copy(vh.at[base + k], vb[k], semv[k]).start()

        @pl.loop(0, _RPW // 2)
        def _pair(p):
            r0 = p * 2
            for k in (0, 1):
                r = r0 + k
                row = base + r

                @pl.when(r >= 2)
                def _reclaim(_k=k, _row=row):
                    pltpu.make_async_copy(acc[_k], skh.at[_row - 2],
                                          sema[_k]).wait()
                    zero_touched(acc[_k], hv)

                pltpu.make_async_copy(vh.at[row], vb[k], semv[k]).wait()

                @pl.loop(0, _N // _L, unroll=8)
                def _scat(j, _k=k):
                    idx = hv[pl.ds(j * _L, _L)]
                    val = vb[_k][pl.ds(j * _L, _L)] * sv[pl.ds(j * _L, _L)]
                    plsc.addupdate_scatter(acc[_k], [idx], val)

                pltpu.make_async_copy(acc[k], skh.at[row], sema[k]).start()

                @pl.when(r + 2 < _RPW)
                def _prefetch(_k=k, _row=row):
                    pltpu.make_async_copy(vh.at[_row + 2], vb[_k],
                                          semv[_k]).start()

        for k in (0, 1):
            pltpu.make_async_copy(acc[k], skh.at[base + _RPW - 2 + k],
                                  sema[k]).wait()
            zero_touched(acc[k], hv)

    phase(v1h, sk1h, hv1, sv1)
    phase(v2h, sk2h, hv2, sv2)


@functools.cache
def _sc_sketch():
    return pl.kernel(
        _sc_body,
        out_type=[jax.ShapeDtypeStruct((_B, _D), jnp.float32),
                  jax.ShapeDtypeStruct((_B, _D), jnp.float32)],
        mesh=plsc.VectorSubcoreMesh(core_axis_name="c", subcore_axis_name="s",
                                    num_cores=_NC, num_subcores=_NS),
        compiler_params=pltpu.CompilerParams(needs_layout_passes=False),
        scratch_types=[
            pltpu.VMEM((_N,), jnp.int32),
            pltpu.VMEM((_N,), jnp.int32),
            pltpu.VMEM((_N,), jnp.float32),
            pltpu.VMEM((_N,), jnp.float32),
            pltpu.VMEM((_N,), jnp.float32),
            pltpu.VMEM((_N,), jnp.float32),
            pltpu.VMEM((_D,), jnp.float32),
            pltpu.VMEM((_D,), jnp.float32),
            pltpu.SemaphoreType.DMA,
            pltpu.SemaphoreType.DMA,
            pltpu.SemaphoreType.DMA,
            pltpu.SemaphoreType.DMA,
        ],
    )


def _conv_body(x1_ref, x2_ref, f1r_ref, f1i_ref, f2r_ref, f2i_ref,
               twr_ref, twi_ref, y_ref, p_ref):
    f1r = f1r_ref[...]
    f1i = f1i_ref[...]
    f2r = f2r_ref[...]
    f2i = f2i_ref[...]
    twr = twr_ref[...][:, None, :]
    twi = twi_ref[...][:, None, :]

    def fwd(x):
        xt = jnp.transpose(x, (1, 0, 2)).reshape(_N1, _R * _N2)
        yr = jnp.dot(f1r, xt, preferred_element_type=jnp.float32)
        yi = jnp.dot(f1i, xt, preferred_element_type=jnp.float32)
        yr = yr.reshape(_N1, _R, _N2)
        yi = yi.reshape(_N1, _R, _N2)
        zr = (yr * twr - yi * twi).reshape(_N1 * _R, _N2)
        zi = (yr * twi + yi * twr).reshape(_N1 * _R, _N2)
        ur = jnp.dot(zr, f2r) - jnp.dot(zi, f2i)
        ui = jnp.dot(zr, f2i) + jnp.dot(zi, f2r)
        return ur, ui

    u1r, u1i = fwd(x1_ref[...])
    u2r, u2i = fwd(x2_ref[...])
    pr = u1r * u2r - u1i * u2i
    pi = u1r * u2i + u1i * u2r
    sr = jnp.dot(pr, f2r) + jnp.dot(pi, f2i)
    si = jnp.dot(pi, f2r) - jnp.dot(pr, f2i)
    sr = sr.reshape(_N1, _R, _N2)
    si = si.reshape(_N1, _R, _N2)
    tr = (sr * twr + si * twi).reshape(_N1, _R * _N2)
    ti = (si * twr - sr * twi).reshape(_N1, _R * _N2)
    xo = (jnp.dot(f1r, tr) + jnp.dot(f1i, ti)) * (1.0 / _D)
    x = jnp.transpose(xo.reshape(_N1, _R, _N2), (1, 0, 2))
    ax = jnp.abs(x)
    y_ref[...] = jnp.sign(x) * jnp.sqrt(ax)
    p_ref[...] = jnp.full((1, 1, 128), jnp.sum(ax) * (1.0 / 128.0), jnp.float32)


_conv = pl.pallas_call(
    _conv_body,
    grid=(_G,),
    in_specs=[
        pl.BlockSpec((_R, _N1, _N2), lambda g: (g, 0, 0)),
        pl.BlockSpec((_R, _N1, _N2), lambda g: (g, 0, 0)),
        pl.BlockSpec((_N1, _N1), lambda g: (0, 0)),
        pl.BlockSpec((_N1, _N1), lambda g: (0, 0)),
        pl.BlockSpec((_N2, _N2), lambda g: (0, 0)),
        pl.BlockSpec((_N2, _N2), lambda g: (0, 0)),
        pl.BlockSpec((_N1, _N2), lambda g: (0, 0)),
        pl.BlockSpec((_N1, _N2), lambda g: (0, 0)),
    ],
    out_specs=[
        pl.BlockSpec((_R, _N1, _N2), lambda g: (g, 0, 0)),
        pl.BlockSpec((1, 1, 128), lambda g: (g, 0, 0)),
    ],
    out_shape=[jax.ShapeDtypeStruct((_B, _N1, _N2), jnp.float32),
               jax.ShapeDtypeStruct((_G, 1, 128), jnp.float32)],
)


def _scale_body(y_ref, p_ref, o_ref):
    total = jnp.sum(p_ref[...])
    norm = jnp.sqrt(jnp.maximum(total, 1e-12))
    o_ref[...] = y_ref[...] * (1.0 / norm)


_scale = pl.pallas_call(
    _scale_body,
    grid=(_G,),
    in_specs=[
        pl.BlockSpec((_R, _D), lambda g: (g, 0)),
        pl.BlockSpec((_G, 1, 128), lambda g: (0, 0, 0)),
    ],
    out_specs=pl.BlockSpec((_R, _D), lambda g: (g, 0)),
    out_shape=jax.ShapeDtypeStruct((_B, _D), jnp.float32),
)


def kernel(v1, v2, s1, s2, h1, h2):
    sk1, sk2 = _sc_sketch()(v1, v2, s1, s2, h1, h2)
    y, parts = _conv(
        sk1.reshape(_B, _N1, _N2), sk2.reshape(_B, _N1, _N2),
        jnp.asarray(_F1R), jnp.asarray(_F1I),
        jnp.asarray(_F2R), jnp.asarray(_F2I),
        jnp.asarray(_TWR), jnp.asarray(_TWI),
    )
    return _scale(y.reshape(_B, _D), parts)


# speedup vs baseline: 5.3349x; 1.3073x over previous
import functools

import numpy as np
import jax
import jax.numpy as jnp
from jax import lax
from jax.experimental import pallas as pl
from jax.experimental.pallas import tpu as pltpu
from jax.experimental.pallas import tpu_sc as plsc

_B, _N, _D = 4096, 2048, 8192
_N1, _N2 = 64, 128
_NC, _NS = 2, 16
_NW = _NC * _NS
_RPW = _B // _NW
_L = 16

_R = 128
_G = _B // _R


def _dft(n):
    k = np.arange(n)
    ang = -2.0 * np.pi * np.outer(k, k) / n
    return np.cos(ang).astype(np.float32), np.sin(ang).astype(np.float32)


_F1R, _F1I = _dft(_N1)
_F2R, _F2I = _dft(_N2)
_F1S = np.vstack([_F1R, _F1I])
_F1C = np.hstack([_F1R, _F1I])
_G2F = np.block([[_F2R, _F2I], [-_F2I, _F2R]])
_G2B = np.block([[_F2R, -_F2I], [_F2I, _F2R]])
_ang = -2.0 * np.pi * np.outer(np.arange(_N1), np.arange(_N2)) / _D
_TWR = np.cos(_ang).astype(np.float32)
_TWI = np.sin(_ang).astype(np.float32)


def _sc_body(v1h, v2h, s1h, s2h, h1h, h2h, sk1h, sk2h,
             hv1, hv2, sv1, sv2, ha1, hb1, ha2, hb2,
             vb0, vb1, acc0, acc1,
             semv0, semv1, sema0, sema1):
    wid = lax.axis_index("s") * _NC + lax.axis_index("c")
    base = wid * _RPW
    pltpu.sync_copy(h1h, hv1)
    pltpu.sync_copy(h2h, hv2)
    pltpu.sync_copy(s1h, sv1)
    pltpu.sync_copy(s2h, sv2)
    vb = (vb0, vb1)
    acc = (acc0, acc1)
    semv = (semv0, semv1)
    sema = (sema0, sema1)

    for hv, ha, hb in ((hv1, ha1, hb1), (hv2, ha2, hb2)):
        @pl.loop(0, _N // _L, unroll=8)
        def _split(j, _hv=hv, _ha=ha, _hb=hb):
            h = _hv[pl.ds(j * _L, _L)]
            _ha[pl.ds(j * _L, _L)] = lax.shift_right_logical(h, 7)
            _hb[pl.ds(j * _L, _L)] = lax.bitwise_and(h, 127)

    for k in (0, 1):
        @pl.loop(0, _N1, unroll=4)
        def _z0(i, _k=k):
            for j in range(_N2 // _L):
                acc[_k][i, pl.ds(j * _L, _L)] = jnp.zeros((_L,), jnp.float32)

    def zero_touched(accr, ha, hb):
        @pl.loop(0, _N // _L, unroll=8)
        def _z(j):
            ia = ha[pl.ds(j * _L, _L)]
            ib = hb[pl.ds(j * _L, _L)]
            plsc.store_scatter(accr, [ia, ib], jnp.zeros((_L,), jnp.float32))

    def phase(vh, skh, ha, hb, sv):
        for k in (0, 1):
            pltpu.make_async_copy(vh.at[base + k], vb[k], semv[k]).start()

        @pl.loop(0, _RPW // 2)
        def _pair(p):
            r0 = p * 2
            for k in (0, 1):
                r = r0 + k
                row = base + r

                @pl.when(r >= 2)
                def _reclaim(_k=k, _row=row):
                    pltpu.make_async_copy(
                        acc[_k], skh.at[:, pl.ds((_row - 2) * _N2, _N2)],
                        sema[_k]).wait()
                    zero_touched(acc[_k], ha, hb)

                pltpu.make_async_copy(vh.at[row], vb[k], semv[k]).wait()

                @pl.loop(0, _N // _L, unroll=8)
                def _scat(j, _k=k):
                    ia = ha[pl.ds(j * _L, _L)]
                    ib = hb[pl.ds(j * _L, _L)]
                    val = vb[_k][pl.ds(j * _L, _L)] * sv[pl.ds(j * _L, _L)]
                    plsc.addupdate_scatter(acc[_k], [ia, ib], val)

                pltpu.make_async_copy(acc[k], skh.at[:, pl.ds(row * _N2, _N2)],
                                      sema[k]).start()

                @pl.when(r + 2 < _RPW)
                def _prefetch(_k=k, _row=row):
                    pltpu.make_async_copy(vh.at[_row + 2], vb[_k],
                                          semv[_k]).start()

        for k in (0, 1):
            row = base + _RPW - 2 + k
            pltpu.make_async_copy(acc[k], skh.at[:, pl.ds(row * _N2, _N2)],
                                  sema[k]).wait()
            zero_touched(acc[k], ha, hb)

    phase(v1h, sk1h, ha1, hb1, sv1)
    phase(v2h, sk2h, ha2, hb2, sv2)


@functools.cache
def _sc_sketch():
    return pl.kernel(
        _sc_body,
        out_type=[jax.ShapeDtypeStruct((_N1, _B * _N2), jnp.float32),
                  jax.ShapeDtypeStruct((_N1, _B * _N2), jnp.float32)],
        mesh=plsc.VectorSubcoreMesh(core_axis_name="c", subcore_axis_name="s",
                                    num_cores=_NC, num_subcores=_NS),
        compiler_params=pltpu.CompilerParams(needs_layout_passes=False),
        scratch_types=[
            pltpu.VMEM((_N,), jnp.int32),
            pltpu.VMEM((_N,), jnp.int32),
            pltpu.VMEM((_N,), jnp.float32),
            pltpu.VMEM((_N,), jnp.float32),
            pltpu.VMEM((_N,), jnp.int32),
            pltpu.VMEM((_N,), jnp.int32),
            pltpu.VMEM((_N,), jnp.int32),
            pltpu.VMEM((_N,), jnp.int32),
            pltpu.VMEM((_N,), jnp.float32),
            pltpu.VMEM((_N,), jnp.float32),
            pltpu.VMEM((_N1, _N2), jnp.float32),
            pltpu.VMEM((_N1, _N2), jnp.float32),
            pltpu.SemaphoreType.DMA,
            pltpu.SemaphoreType.DMA,
            pltpu.SemaphoreType.DMA,
            pltpu.SemaphoreType.DMA,
        ],
    )


def _conv_body(x1_ref, x2_ref, f1s_ref, f1c_ref, g2f_ref, g2b_ref,
               twr_ref, twi_ref, y_ref, p_ref):
    f1s = f1s_ref[...]
    f1c = f1c_ref[...]
    g2f = g2f_ref[...]
    g2b = g2b_ref[...]
    twr = twr_ref[...][:, None, :]
    twi = twi_ref[...][:, None, :]

    def fwd(xt):
        y = jnp.dot(f1s, xt, preferred_element_type=jnp.float32)
        yr = y[:_N1].reshape(_N1, _R, _N2)
        yi = y[_N1:].reshape(_N1, _R, _N2)
        zr = (yr * twr - yi * twi).reshape(_N1 * _R, _N2)
        zi = (yr * twi + yi * twr).reshape(_N1 * _R, _N2)
        zc = jnp.concatenate([zr, zi], axis=1)
        return jnp.dot(zc, g2f, preferred_element_type=jnp.float32)

    u1 = fwd(x1_ref[...])
    u2 = fwd(x2_ref[...])
    u1r, u1i = u1[:, :_N2], u1[:, _N2:]
    u2r, u2i = u2[:, :_N2], u2[:, _N2:]
    pc = jnp.concatenate([u1r * u2r - u1i * u2i,
                          u1r * u2i + u1i * u2r], axis=1)
    s = jnp.dot(pc, g2b, preferred_element_type=jnp.float32)
    sr = s[:, :_N2].reshape(_N1, _R, _N2)
    si = s[:, _N2:].reshape(_N1, _R, _N2)
    tr = (sr * twr + si * twi).reshape(_N1, _R * _N2)
    ti = (si * twr - sr * twi).reshape(_N1, _R * _N2)
    tc = jnp.concatenate([tr, ti], axis=0)
    xo = jnp.dot(f1c, tc, preferred_element_type=jnp.float32) * (1.0 / _D)
    ax = jnp.abs(xo)
    y_ref[...] = jnp.sign(xo) * jnp.sqrt(ax)
    p_ref[...] = jnp.full((1, 1, 128), jnp.sum(ax) * (1.0 / 128.0), jnp.float32)


_conv = pl.pallas_call(
    _conv_body,
    grid=(_G,),
    in_specs=[
        pl.BlockSpec((_N1, _R * _N2), lambda g: (0, g)),
        pl.BlockSpec((_N1, _R * _N2), lambda g: (0, g)),
        pl.BlockSpec((2 * _N1, _N1), lambda g: (0, 0)),
        pl.BlockSpec((_N1, 2 * _N1), lambda g: (0, 0)),
        pl.BlockSpec((2 * _N2, 2 * _N2), lambda g: (0, 0)),
        pl.BlockSpec((2 * _N2, 2 * _N2), lambda g: (0, 0)),
        pl.BlockSpec((_N1, _N2), lambda g: (0, 0)),
        pl.BlockSpec((_N1, _N2), lambda g: (0, 0)),
    ],
    out_specs=[
        pl.BlockSpec((_N1, _R * _N2), lambda g: (0, g)),
        pl.BlockSpec((1, 1, 128), lambda g: (g, 0, 0)),
    ],
    out_shape=[jax.ShapeDtypeStruct((_N1, _B * _N2), jnp.float32),
               jax.ShapeDtypeStruct((_G, 1, 128), jnp.float32)],
)


def _scale_body(y_ref, p_ref, o_ref):
    total = jnp.sum(p_ref[...])
    norm = jnp.sqrt(jnp.maximum(total, 1e-12))
    y = y_ref[...] * (1.0 / norm)
    y = y.reshape(_N1, _R, _N2).transpose(1, 0, 2)
    o_ref[...] = y.reshape(_R, _D)


_scale = pl.pallas_call(
    _scale_body,
    grid=(_G,),
    in_specs=[
        pl.BlockSpec((_N1, _R * _N2), lambda g: (0, g)),
        pl.BlockSpec((_G, 1, 128), lambda g: (0, 0, 0)),
    ],
    out_specs=pl.BlockSpec((_R, _D), lambda g: (g, 0)),
    out_shape=jax.ShapeDtypeStruct((_B, _D), jnp.float32),
)


def kernel(v1, v2, s1, s2, h1, h2):
    skt1, skt2 = _sc_sketch()(v1, v2, s1, s2, h1, h2)
    y, parts = _conv(
        skt1, skt2,
        jnp.asarray(_F1S), jnp.asarray(_F1C),
        jnp.asarray(_G2F), jnp.asarray(_G2B),
        jnp.asarray(_TWR), jnp.asarray(_TWI),
    )
    return _scale(y, parts)


# speedup vs baseline: 6.2508x; 1.1717x over previous
import functools

import numpy as np
import jax
import jax.numpy as jnp
from jax import lax
from jax.experimental import pallas as pl
from jax.experimental.pallas import tpu as pltpu
from jax.experimental.pallas import tpu_sc as plsc

_B, _N, _D = 4096, 2048, 8192
_N1, _N2 = 64, 128
_NC, _NS = 2, 16
_NW = _NC * _NS
_RPW = _B // _NW
_L = 16

_R = 128
_G = _B // _R


def _dft(n):
    k = np.arange(n)
    ang = -2.0 * np.pi * np.outer(k, k) / n
    return np.cos(ang).astype(np.float32), np.sin(ang).astype(np.float32)


_F1R, _F1I = _dft(_N1)
_F2R, _F2I = _dft(_N2)
_F1S = np.vstack([_F1R, _F1I])
_F1C = np.hstack([_F1R, _F1I])
_G2F = np.block([[_F2R, _F2I], [-_F2I, _F2R]])
_G2B = np.block([[_F2R, -_F2I], [_F2I, _F2R]])
_ang = -2.0 * np.pi * np.outer(np.arange(_N1), np.arange(_N2)) / _D
_TWR = np.cos(_ang).astype(np.float32)
_TWI = np.sin(_ang).astype(np.float32)


def _sc_body(v1h, v2h, s1h, s2h, h1h, h2h, sk1h, sk2h,
             hv1, hv2, sv1, sv2,
             vb0, vb1, acc0, acc1,
             semv0, semv1, sema0, sema1):
    wid = lax.axis_index("s") * _NC + lax.axis_index("c")
    base = wid * _RPW
    pltpu.sync_copy(h1h, hv1)
    pltpu.sync_copy(h2h, hv2)
    pltpu.sync_copy(s1h, sv1)
    pltpu.sync_copy(s2h, sv2)
    vb = (vb0, vb1)
    acc = (acc0, acc1)
    semv = (semv0, semv1)
    sema = (sema0, sema1)

    zv = jnp.zeros((_L,), jnp.int32)

    for k in (0, 1):
        @pl.loop(0, _N1, unroll=4)
        def _z0(i, _k=k):
            for j in range(_N2 // _L):
                acc[_k][i, pl.ds(j * _L, _L)] = jnp.zeros((_L,), jnp.float32)

    def zero_touched(accr, hv):
        @pl.loop(0, _N // _L, unroll=8)
        def _z(j):
            idx = hv[pl.ds(j * _L, _L)]
            plsc.store_scatter(accr, [zv, idx], jnp.zeros((_L,), jnp.float32))

    def phase(vh, skh, hv, sv):
        for k in (0, 1):
            pltpu.make_async_copy(vh.at[base + k], vb[k], semv[k]).start()

        @pl.loop(0, _RPW // 2)
        def _pair(p):
            r0 = p * 2
            for k in (0, 1):
                r = r0 + k
                row = base + r

                @pl.when(r >= 2)
                def _reclaim(_k=k, _row=row):
                    pltpu.make_async_copy(
                        acc[_k],
                        skh.at[:, pl.ds((_row - 2) * _N2, _N2)],
                        sema[_k]).wait()
                    zero_touched(acc[_k], hv)

                pltpu.make_async_copy(vh.at[row], vb[k], semv[k]).wait()

                @pl.loop(0, _N // _L, unroll=8)
                def _scat(j, _k=k):
                    idx = hv[pl.ds(j * _L, _L)]
                    val = vb[_k][pl.ds(j * _L, _L)] * sv[pl.ds(j * _L, _L)]
                    plsc.addupdate_scatter(acc[_k], [zv, idx], val)

                pltpu.make_async_copy(acc[k],
                                      skh.at[:, pl.ds(row * _N2, _N2)],
                                      sema[k]).start()

                @pl.when(r + 2 < _RPW)
                def _prefetch(_k=k, _row=row):
                    pltpu.make_async_copy(vh.at[_row + 2], vb[_k],
                                          semv[_k]).start()

        for k in (0, 1):
            row = base + _RPW - 2 + k
            pltpu.make_async_copy(acc[k],
                                  skh.at[:, pl.ds(row * _N2, _N2)],
                                  sema[k]).wait()
            zero_touched(acc[k], hv)

    phase(v1h, sk1h, hv1, sv1)
    phase(v2h, sk2h, hv2, sv2)


@functools.cache
def _sc_sketch():
    return pl.kernel(
        _sc_body,
        out_type=[jax.ShapeDtypeStruct((_N1, _B * _N2), jnp.float32),
                  jax.ShapeDtypeStruct((_N1, _B * _N2), jnp.float32)],
        mesh=plsc.VectorSubcoreMesh(core_axis_name="c", subcore_axis_name="s",
                                    num_cores=_NC, num_subcores=_NS),
        compiler_params=pltpu.CompilerParams(needs_layout_passes=False),
        scratch_types=[
            pltpu.VMEM((_N,), jnp.int32),
            pltpu.VMEM((_N,), jnp.int32),
            pltpu.VMEM((_N,), jnp.float32),
            pltpu.VMEM((_N,), jnp.float32),
            pltpu.VMEM((_N,), jnp.float32),
            pltpu.VMEM((_N,), jnp.float32),
            pltpu.VMEM((_N1, _N2), jnp.float32),
            pltpu.VMEM((_N1, _N2), jnp.float32),
            pltpu.SemaphoreType.DMA,
            pltpu.SemaphoreType.DMA,
            pltpu.SemaphoreType.DMA,
            pltpu.SemaphoreType.DMA,
        ],
    )


def _conv_body(x1_ref, x2_ref, f1s_ref, f1c_ref, g2f_ref, g2b_ref,
               twr_ref, twi_ref, y_ref, p_ref):
    f1s = f1s_ref[...]
    f1c = f1c_ref[...]
    g2f = g2f_ref[...]
    g2b = g2b_ref[...]
    twr = twr_ref[...][:, None, :]
    twi = twi_ref[...][:, None, :]

    def fwd(xt):
        y = jnp.dot(f1s, xt, preferred_element_type=jnp.float32)
        yr = y[:_N1].reshape(_N1, _R, _N2)
        yi = y[_N1:].reshape(_N1, _R, _N2)
        zr = (yr * twr - yi * twi).reshape(_N1 * _R, _N2)
        zi = (yr * twi + yi * twr).reshape(_N1 * _R, _N2)
        zc = jnp.concatenate([zr, zi], axis=1)
        return jnp.dot(zc, g2f, preferred_element_type=jnp.float32)

    u1 = fwd(x1_ref[...])
    u2 = fwd(x2_ref[...])
    u1r, u1i = u1[:, :_N2], u1[:, _N2:]
    u2r, u2i = u2[:, :_N2], u2[:, _N2:]
    pc = jnp.concatenate([u1r * u2r - u1i * u2i,
                          u1r * u2i + u1i * u2r], axis=1)
    s = jnp.dot(pc, g2b, preferred_element_type=jnp.float32)
    sr = s[:, :_N2].reshape(_N1, _R, _N2)
    si = s[:, _N2:].reshape(_N1, _R, _N2)
    tr = (sr * twr + si * twi).reshape(_N1, _R * _N2)
    ti = (si * twr - sr * twi).reshape(_N1, _R * _N2)
    tc = jnp.concatenate([tr, ti], axis=0)
    xo = jnp.dot(f1c, tc, preferred_element_type=jnp.float32) * (1.0 / _D)
    ax = jnp.abs(xo)
    y_ref[...] = jnp.sign(xo) * jnp.sqrt(ax)
    p_ref[...] = jnp.full((1, 1, 128), jnp.sum(ax) * (1.0 / 128.0), jnp.float32)


_conv = pl.pallas_call(
    _conv_body,
    grid=(_G,),
    in_specs=[
        pl.BlockSpec((_N1, _R * _N2), lambda g: (0, g)),
        pl.BlockSpec((_N1, _R * _N2), lambda g: (0, g)),
        pl.BlockSpec((2 * _N1, _N1), lambda g: (0, 0)),
        pl.BlockSpec((_N1, 2 * _N1), lambda g: (0, 0)),
        pl.BlockSpec((2 * _N2, 2 * _N2), lambda g: (0, 0)),
        pl.BlockSpec((2 * _N2, 2 * _N2), lambda g: (0, 0)),
        pl.BlockSpec((_N1, _N2), lambda g: (0, 0)),
        pl.BlockSpec((_N1, _N2), lambda g: (0, 0)),
    ],
    out_specs=[
        pl.BlockSpec((_N1, _R * _N2), lambda g: (0, g)),
        pl.BlockSpec((1, 1, 128), lambda g: (g, 0, 0)),
    ],
    out_shape=[jax.ShapeDtypeStruct((_N1, _B * _N2), jnp.float32),
               jax.ShapeDtypeStruct((_G, 1, 128), jnp.float32)],
)


def _scale_body(y_ref, p_ref, o_ref):
    total = jnp.sum(p_ref[...])
    norm = jnp.sqrt(jnp.maximum(total, 1e-12))
    y = y_ref[...] * (1.0 / norm)
    y = y.reshape(_N1, _R, _N2).transpose(1, 0, 2)
    o_ref[...] = y.reshape(_R, _D)


_scale = pl.pallas_call(
    _scale_body,
    grid=(_G,),
    in_specs=[
        pl.BlockSpec((_N1, _R * _N2), lambda g: (0, g)),
        pl.BlockSpec((_G, 1, 128), lambda g: (0, 0, 0)),
    ],
    out_specs=pl.BlockSpec((_R, _D), lambda g: (g, 0)),
    out_shape=jax.ShapeDtypeStruct((_B, _D), jnp.float32),
)


def kernel(v1, v2, s1, s2, h1, h2):
    skt1, skt2 = _sc_sketch()(v1, v2, s1, s2, h1, h2)
    y, parts = _conv(
        skt1, skt2,
        jnp.asarray(_F1S), jnp.asarray(_F1C),
        jnp.asarray(_G2F), jnp.asarray(_G2B),
        jnp.asarray(_TWR), jnp.asarray(_TWI),
    )
    return _scale(y, parts)
